# Initial kernel scaffold; baseline (speedup 1.0000x reference)
import jax
import jax.numpy as jnp
from jax.experimental import pallas as pl


def kernel(x, W, b):
    raise NotImplementedError("write your pallas kernel here")


# speedup vs baseline: 19.0633x; 19.0633x over previous
import functools

import jax
import jax.numpy as jnp
from jax.experimental import pallas as pl

K = 3
R = 256


def _nn_conv_kernel(x_ref, xt_ref, wt_ref, b_ref, o_ref, *, n_tokens):
    xb = x_ref[0]
    C, N = xb.shape
    nsq = jnp.sum(xb * xb, axis=0, keepdims=True)
    xt = xt_ref[0]
    nsq_t = jnp.sum(xt * xt, axis=0, keepdims=True)

    dot = jax.lax.dot_general(
        xt, xb, (((0,), (0,)), ((), ())),
        preferred_element_type=jnp.float32,
        precision=jax.lax.Precision.DEFAULT)
    d2 = nsq_t.T + nsq - 2.0 * dot
    d = jnp.sqrt(jnp.maximum(d2, 0.0))

    iota_r = jax.lax.broadcasted_iota(jnp.int32, (R, N), 1)
    iota_c = jax.lax.broadcasted_iota(jnp.int32, (N, R), 0)

    acc = jnp.zeros((C, R), dtype=jnp.float32)
    for k in range(K):
        mval = jnp.min(d, axis=1, keepdims=True)
        jidx = jnp.min(jnp.where(d == mval, iota_r, n_tokens),
                       axis=1, keepdims=True)
        maskT = (iota_c == jidx.T).astype(jnp.float32)
        g = jax.lax.dot_general(
            xb, maskT, (((1,), (0,)), ((), ())),
            preferred_element_type=jnp.float32,
            precision=jax.lax.Precision.HIGHEST)
        acc = acc + jax.lax.dot_general(
            wt_ref[k], g, (((1,), (0,)), ((), ())),
            preferred_element_type=jnp.float32,
            precision=jax.lax.Precision.HIGHEST)
        d = jnp.where(iota_r == jidx, jnp.inf, d)
    o_ref[0] = jnp.maximum(acc + b_ref[:], 0.0)


def kernel(x, W, b):
    B, C, H, Wd = x.shape
    N = H * Wd
    x1 = x.reshape(B, C, N)
    Wt = jnp.transpose(W, (2, 0, 1))
    b2 = b.reshape(C, 1)

    out = pl.pallas_call(
        functools.partial(_nn_conv_kernel, n_tokens=N),
        grid=(B, N // R),
        in_specs=[
            pl.BlockSpec((1, C, N), lambda bb, ii: (bb, 0, 0)),
            pl.BlockSpec((1, C, R), lambda bb, ii: (bb, 0, ii)),
            pl.BlockSpec((K, C, C), lambda bb, ii: (0, 0, 0)),
            pl.BlockSpec((C, 1), lambda bb, ii: (0, 0)),
        ],
        out_specs=pl.BlockSpec((1, C, R), lambda bb, ii: (bb, 0, ii)),
        out_shape=jax.ShapeDtypeStruct((B, C, N), jnp.float32),
    )(x1, x1, Wt, b2)
    return out.reshape(B, C, H, Wd)


# speedup vs baseline: 38.1394x; 2.0007x over previous
import functools

import jax
import jax.numpy as jnp
from jax.experimental import pallas as pl

K = 3
R = 256


def _nn_conv_kernel(x_ref, xt_ref, wt_ref, b_ref, o_ref, *, n_tokens):
    xb = x_ref[0]
    C, N = xb.shape
    nsq = jnp.sum(xb * xb, axis=0, keepdims=True)
    xt = xt_ref[0]
    nsq_t = jnp.sum(xt * xt, axis=0, keepdims=True)

    dot = jax.lax.dot_general(
        xt, xb, (((0,), (0,)), ((), ())),
        preferred_element_type=jnp.float32,
        precision=jax.lax.Precision.DEFAULT)
    d2 = nsq_t.T + nsq - 2.0 * dot
    d = jnp.sqrt(jnp.maximum(d2, 0.0))

    iota_r = jax.lax.broadcasted_iota(jnp.int32, (R, N), 1)
    iota_c = jax.lax.broadcasted_iota(jnp.int32, (N, R), 0)

    acc = jnp.zeros((C, R), dtype=jnp.float32)
    for k in range(K):
        mval = jnp.min(d, axis=1, keepdims=True)
        jidx = jnp.min(jnp.where(d == mval, iota_r, n_tokens),
                       axis=1, keepdims=True)
        maskT = (iota_c == jidx.T).astype(jnp.float32)
        g = jax.lax.dot_general(
            xb, maskT, (((1,), (0,)), ((), ())),
            preferred_element_type=jnp.float32,
            precision=jax.lax.Precision.DEFAULT)
        acc = acc + jax.lax.dot_general(
            wt_ref[k], g, (((1,), (0,)), ((), ())),
            preferred_element_type=jnp.float32,
            precision=jax.lax.Precision.DEFAULT)
        d = jnp.where(iota_r == jidx, jnp.inf, d)
    o_ref[0] = jnp.maximum(acc + b_ref[:], 0.0)


def kernel(x, W, b):
    B, C, H, Wd = x.shape
    N = H * Wd
    x1 = x.reshape(B, C, N)
    Wt = jnp.transpose(W, (2, 0, 1))
    b2 = b.reshape(C, 1)

    out = pl.pallas_call(
        functools.partial(_nn_conv_kernel, n_tokens=N),
        grid=(B, N // R),
        in_specs=[
            pl.BlockSpec((1, C, N), lambda bb, ii: (bb, 0, 0)),
            pl.BlockSpec((1, C, R), lambda bb, ii: (bb, 0, ii)),
            pl.BlockSpec((K, C, C), lambda bb, ii: (0, 0, 0)),
            pl.BlockSpec((C, 1), lambda bb, ii: (0, 0)),
        ],
        out_specs=pl.BlockSpec((1, C, R), lambda bb, ii: (bb, 0, ii)),
        out_shape=jax.ShapeDtypeStruct((B, C, N), jnp.float32),
    )(x1, x1, Wt, b2)
    return out.reshape(B, C, H, Wd)


# speedup vs baseline: 39.5744x; 1.0376x over previous
import functools

import jax
import jax.numpy as jnp
from jax.experimental import pallas as pl

K = 3
R = 256


def _nn_conv_kernel(x_ref, xt_ref, wt_ref, b_ref, o_ref, *, n_tokens):
    xb = x_ref[0]
    C, N = xb.shape
    nsq = jnp.sum(xb * xb, axis=0, keepdims=True)
    xt = xt_ref[0]
    nsq_t = jnp.sum(xt * xt, axis=0, keepdims=True)

    dot = jax.lax.dot_general(
        xt, xb, (((0,), (0,)), ((), ())),
        preferred_element_type=jnp.float32,
        precision=jax.lax.Precision.DEFAULT)
    d2 = nsq_t.T + nsq - 2.0 * dot
    d = jnp.sqrt(jnp.maximum(d2, 0.0))

    iota_r = jax.lax.broadcasted_iota(jnp.int32, (R, N), 1)
    iota_c = jax.lax.broadcasted_iota(jnp.int32, (N, R), 0)

    acc = jnp.zeros((C, R), dtype=jnp.float32)
    jprev = []
    for k in range(K):
        dk = d
        for jp in jprev:
            dk = jnp.where(iota_r == jp, jnp.inf, dk)
        jidx = jnp.argmin(dk, axis=1).reshape(R, 1)
        jprev.append(jidx)
        maskT = (iota_c == jidx.T).astype(jnp.float32)
        g = jax.lax.dot_general(
            xb, maskT, (((1,), (0,)), ((), ())),
            preferred_element_type=jnp.float32,
            precision=jax.lax.Precision.DEFAULT)
        acc = acc + jax.lax.dot_general(
            wt_ref[k], g, (((1,), (0,)), ((), ())),
            preferred_element_type=jnp.float32,
            precision=jax.lax.Precision.DEFAULT)
    o_ref[0] = jnp.maximum(acc + b_ref[:], 0.0)


def kernel(x, W, b):
    B, C, H, Wd = x.shape
    N = H * Wd
    x1 = x.reshape(B, C, N)
    Wt = jnp.transpose(W, (2, 0, 1))
    b2 = b.reshape(C, 1)

    out = pl.pallas_call(
        functools.partial(_nn_conv_kernel, n_tokens=N),
        grid=(B, N // R),
        in_specs=[
            pl.BlockSpec((1, C, N), lambda bb, ii: (bb, 0, 0)),
            pl.BlockSpec((1, C, R), lambda bb, ii: (bb, 0, ii)),
            pl.BlockSpec((K, C, C), lambda bb, ii: (0, 0, 0)),
            pl.BlockSpec((C, 1), lambda bb, ii: (0, 0)),
        ],
        out_specs=pl.BlockSpec((1, C, R), lambda bb, ii: (bb, 0, ii)),
        out_shape=jax.ShapeDtypeStruct((B, C, N), jnp.float32),
    )(x1, x1, Wt, b2)
    return out.reshape(B, C, H, Wd)


# speedup vs baseline: 47.3160x; 1.1956x over previous
import functools

import jax
import jax.numpy as jnp
from jax.experimental import pallas as pl

K = 3
R = 256


def _nn_conv_kernel(x_ref, xt_ref, wt_ref, b_ref, o_ref, *, n_tokens):
    xb = x_ref[0]
    C, N = xb.shape
    nsq = jnp.sum(xb * xb, axis=0, keepdims=True)
    xt = xt_ref[0]
    nsq_t = jnp.sum(xt * xt, axis=0, keepdims=True)

    dot = jax.lax.dot_general(
        xt, xb, (((0,), (0,)), ((), ())),
        preferred_element_type=jnp.float32,
        precision=jax.lax.Precision.DEFAULT)
    d = jnp.maximum(nsq_t.T + nsq - 2.0 * dot, 0.0)

    iota_r = jax.lax.broadcasted_iota(jnp.int32, (R, N), 1)
    iota_c = jax.lax.broadcasted_iota(jnp.int32, (N, R), 0)

    acc = jnp.zeros((C, R), dtype=jnp.float32)
    jprev = []
    for k in range(K):
        dk = d
        for jp in jprev:
            dk = jnp.where(iota_r == jp, jnp.inf, dk)
        jidx = jnp.argmin(dk, axis=1).reshape(R, 1)
        jprev.append(jidx)
        maskT = (iota_c == jidx.T).astype(jnp.float32)
        g = jax.lax.dot_general(
            xb, maskT, (((1,), (0,)), ((), ())),
            preferred_element_type=jnp.float32,
            precision=jax.lax.Precision.DEFAULT)
        acc = acc + jax.lax.dot_general(
            wt_ref[k], g, (((1,), (0,)), ((), ())),
            preferred_element_type=jnp.float32,
            precision=jax.lax.Precision.DEFAULT)
    o_ref[0] = jnp.maximum(acc + b_ref[:], 0.0)


def kernel(x, W, b):
    B, C, H, Wd = x.shape
    N = H * Wd
    x1 = x.reshape(B, C, N)
    Wt = jnp.transpose(W, (2, 0, 1))
    b2 = b.reshape(C, 1)

    out = pl.pallas_call(
        functools.partial(_nn_conv_kernel, n_tokens=N),
        grid=(B, N // R),
        in_specs=[
            pl.BlockSpec((1, C, N), lambda bb, ii: (bb, 0, 0)),
            pl.BlockSpec((1, C, R), lambda bb, ii: (bb, 0, ii)),
            pl.BlockSpec((K, C, C), lambda bb, ii: (0, 0, 0)),
            pl.BlockSpec((C, 1), lambda bb, ii: (0, 0)),
        ],
        out_specs=pl.BlockSpec((1, C, R), lambda bb, ii: (bb, 0, ii)),
        out_shape=jax.ShapeDtypeStruct((B, C, N), jnp.float32),
    )(x1, x1, Wt, b2)
    return out.reshape(B, C, H, Wd)


# speedup vs baseline: 51.8299x; 1.0954x over previous
import functools

import jax
import jax.numpy as jnp
from jax import lax
from jax.experimental import pallas as pl
from jax.experimental.pallas import tpu as pltpu
from jax.experimental.pallas import tpu_sc as plsc

K = 3
R = 256
CHUNK = 128


def _topk_kernel(x_ref, xt_ref, wt_ref, idx_ref, yt_ref, *, n_tokens):
    xb = x_ref[0]
    C, N = xb.shape
    nsq = jnp.sum(xb * xb, axis=0, keepdims=True)
    xt = xt_ref[0]
    nsq_t = jnp.sum(xt * xt, axis=0, keepdims=True)

    dot = jax.lax.dot_general(
        xt, xb, (((0,), (0,)), ((), ())),
        preferred_element_type=jnp.float32,
        precision=jax.lax.Precision.DEFAULT)
    d = jnp.maximum(nsq_t.T + nsq - 2.0 * dot, 0.0)

    iota_r = jax.lax.broadcasted_iota(jnp.int32, (R, N), 1)
    b = pl.program_id(0)
    jprev = []
    for k in range(K):
        dk = d
        for jp in jprev:
            dk = jnp.where(iota_r == jp, jnp.inf, dk)
        jidx = jnp.argmin(dk, axis=1).reshape(R, 1)
        jprev.append(jidx)
        idx_ref[0, k] = (jidx + (b * K + k) * n_tokens).reshape(R)
        yt_ref[0, k] = jax.lax.dot_general(
            xt, wt_ref[k], (((0,), (1,)), ((), ())),
            preferred_element_type=jnp.float32,
            precision=jax.lax.Precision.DEFAULT)


CP = 128


def _make_sc_gather(B, N, C):
    n_tok = B * N
    info = plsc.get_sparse_core_info()
    NC, NS = info.num_cores, info.num_subcores
    NW = NC * NS
    per_w = n_tok // NW
    n_rounds = per_w // CHUNK

    @functools.partial(
        pl.kernel,
        out_type=jax.ShapeDtypeStruct((n_tok, C), jnp.float32),
        mesh=plsc.VectorSubcoreMesh(core_axis_name="c", subcore_axis_name="s"),
        scratch_types=[
            pltpu.VMEM((CHUNK,), jnp.int32),
            pltpu.VMEM((CHUNK,), jnp.int32),
            pltpu.VMEM((CHUNK,), jnp.int32),
            pltpu.VMEM((CHUNK, CP), jnp.float32),
            pltpu.VMEM((CHUNK, CP), jnp.float32),
            pltpu.VMEM((CHUNK, CP), jnp.float32),
            pltpu.VMEM((CHUNK, C), jnp.float32),
            pltpu.VMEM((C,), jnp.float32),
            pltpu.SemaphoreType.DMA,
            pltpu.SemaphoreType.DMA,
            pltpu.SemaphoreType.DMA,
        ],
    )
    def sc_gather(ytab_hbm, idx_hbm, bias_hbm, out_hbm,
                  i0, i1, i2, r0, r1, r2, ov, bv, s0, s1, s2):
        wid = lax.axis_index("s") * NC + lax.axis_index("c")
        base = wid * per_w
        pltpu.sync_copy(bias_hbm, bv)
        idx_refs = (i0, i1, i2)
        row_refs = (r0, r1, r2)
        sems = (s0, s1, s2)
        for c in range(n_rounds):
            t0 = base + c * CHUNK
            copies = []
            for k in range(K):
                pltpu.sync_copy(idx_hbm.at[pl.ds(k * n_tok + t0, CHUNK)],
                                idx_refs[k])
                copies.append(pltpu.async_copy(ytab_hbm.at[idx_refs[k]],
                                               row_refs[k], sems[k]))
            for cp in copies:
                cp.wait()

            def body(t, carry):
                for cc in range(C // 16):
                    sl = pl.ds(cc * 16, 16)
                    v = r0[t, sl] + r1[t, sl] + r2[t, sl] + bv[sl]
                    ov[t, sl] = jnp.maximum(v, 0.0)
                return carry

            lax.fori_loop(0, CHUNK, body, 0)
            pltpu.sync_copy(ov, out_hbm.at[pl.ds(t0, CHUNK)])

    return sc_gather


def kernel(x, W, b):
    B, C, H, Wd = x.shape
    N = H * Wd
    x1 = x.reshape(B, C, N)
    Wt = jnp.transpose(W, (2, 0, 1))
    Wtp = jnp.zeros((K, CP, C), jnp.float32).at[:, :C, :].set(Wt)

    idx, yt = pl.pallas_call(
        functools.partial(_topk_kernel, n_tokens=N),
        grid=(B, N // R),
        in_specs=[
            pl.BlockSpec((1, C, N), lambda bb, ii: (bb, 0, 0)),
            pl.BlockSpec((1, C, R), lambda bb, ii: (bb, 0, ii)),
            pl.BlockSpec((K, CP, C), lambda bb, ii: (0, 0, 0)),
        ],
        out_specs=[
            pl.BlockSpec((1, K, R), lambda bb, ii: (bb, 0, ii)),
            pl.BlockSpec((1, K, R, CP), lambda bb, ii: (bb, 0, ii, 0)),
        ],
        out_shape=[
            jax.ShapeDtypeStruct((B, K, N), jnp.int32),
            jax.ShapeDtypeStruct((B, K, N, CP), jnp.float32),
        ],
    )(x1, x1, Wtp)

    ytab = yt.reshape(B * K * N, CP)
    idxf = jnp.transpose(idx, (1, 0, 2)).reshape(K * B * N)
    outf = _make_sc_gather(B, N, C)(ytab, idxf, b)
    return jnp.transpose(outf.reshape(B, H, Wd, C), (0, 3, 1, 2))
